# Initial kernel scaffold; baseline (speedup 1.0000x reference)
import jax
import jax.numpy as jnp
from jax.experimental import pallas as pl


def kernel(x, edge_index, W1, b1, W2, b2):
    raise NotImplementedError("write your pallas kernel here")


# speedup vs baseline: 15.6701x; 15.6701x over previous
import functools

import jax
import jax.numpy as jnp
from jax import lax
from jax.experimental import pallas as pl
from jax.experimental.pallas import tpu as pltpu
from jax.experimental.pallas import tpu_sc as plsc

NN = 10000
FD = 128
NE = 320000

NPAD = 10240
RPT = NPAD // 16
NW = 32
CHUNK = 128
NCH = 81
EPT = NCH * CHUNK
EPAD = NW * EPT
BLK = 1280


def _sc_deg_body(dst_hbm, ones_hbm, zero_hbm, out_hbm, dst_v, ones_v, acc):
    cid = lax.axis_index("c")
    sid = lax.axis_index("s")
    wid = cid * 16 + sid
    pltpu.sync_copy(zero_hbm, acc.at[pl.ds(sid * RPT, RPT)])
    pltpu.sync_copy(dst_hbm.at[wid], dst_v)
    pltpu.sync_copy(ones_hbm, ones_v)
    plsc.subcore_barrier()

    def body(j, carry):
        pltpu.sync_copy(ones_v, acc.at[dst_v.at[j]], add=True)
        return carry

    lax.fori_loop(0, NCH, body, 0)
    plsc.subcore_barrier()
    pltpu.sync_copy(acc.at[pl.ds(sid * RPT, RPT)],
                    out_hbm.at[cid, pl.ds(sid * RPT, RPT)])


_sc_deg = functools.partial(
    pl.kernel,
    out_type=jax.ShapeDtypeStruct((2, NPAD, FD), jnp.float32),
    mesh=plsc.VectorSubcoreMesh(core_axis_name="c", subcore_axis_name="s"),
    scratch_types=[
        pltpu.VMEM((NCH, CHUNK), jnp.int32),
        pltpu.VMEM((CHUNK, FD), jnp.float32),
        pltpu.VMEM_SHARED((NPAD, FD), jnp.float32),
    ],
)(_sc_deg_body)


def _sc_agg_body(hh_hbm, src_hbm, dst_hbm, zero_hbm, out_hbm,
                 src_v, dst_v, rows_v, acc, sem):
    cid = lax.axis_index("c")
    sid = lax.axis_index("s")
    wid = cid * 16 + sid
    pltpu.sync_copy(zero_hbm, acc.at[pl.ds(sid * RPT, RPT)])
    pltpu.sync_copy(src_hbm.at[wid], src_v)
    pltpu.sync_copy(dst_hbm.at[wid], dst_v)
    plsc.subcore_barrier()

    def body(j, carry):
        pltpu.async_copy(hh_hbm.at[src_v.at[j]], rows_v, sem).wait()
        pltpu.sync_copy(rows_v, acc.at[dst_v.at[j]], add=True)
        return carry

    lax.fori_loop(0, NCH, body, 0)
    plsc.subcore_barrier()
    pltpu.sync_copy(acc.at[pl.ds(sid * RPT, RPT)],
                    out_hbm.at[cid, pl.ds(sid * RPT, RPT)])


_sc_agg = functools.partial(
    pl.kernel,
    out_type=jax.ShapeDtypeStruct((2, NPAD, FD), jnp.float32),
    mesh=plsc.VectorSubcoreMesh(core_axis_name="c", subcore_axis_name="s"),
    scratch_types=[
        pltpu.VMEM((NCH, CHUNK), jnp.int32),
        pltpu.VMEM((NCH, CHUNK), jnp.int32),
        pltpu.VMEM((CHUNK, FD), jnp.float32),
        pltpu.VMEM_SHARED((NPAD, FD), jnp.float32),
        pltpu.SemaphoreType.DMA,
    ],
)(_sc_agg_body)


def _tc1_body(deg_ref, x_ref, w_ref, hh_ref, dinv_ref):
    d = deg_ref[0, :, 0:1] + deg_ref[1, :, 0:1]
    dinv = jnp.where(d > 0, lax.rsqrt(jnp.maximum(d, 1e-12)), 0.0)
    h = jnp.dot(x_ref[...], w_ref[...], preferred_element_type=jnp.float32)
    hh_ref[...] = h * dinv
    dinv_ref[...] = jnp.broadcast_to(dinv, (BLK, FD))


def _tc2_body(ap_ref, dinv_ref, b_ref, w_ref, hh_ref):
    a = ap_ref[0] + ap_ref[1]
    o = jnp.maximum(dinv_ref[...] * a + b_ref[...], 0.0)
    h = jnp.dot(o, w_ref[...], preferred_element_type=jnp.float32)
    hh_ref[...] = h * dinv_ref[...]


def _tc3_body(ap_ref, dinv_ref, b_ref, out_ref):
    a = ap_ref[0] + ap_ref[1]
    out_ref[...] = jnp.maximum(dinv_ref[...] * a + b_ref[...], 0.0)


def _tc1(deg_p, x_pad, w1):
    grid = NPAD // BLK
    return pl.pallas_call(
        _tc1_body,
        grid=(grid,),
        in_specs=[
            pl.BlockSpec((2, BLK, FD), lambda i: (0, i, 0)),
            pl.BlockSpec((BLK, FD), lambda i: (i, 0)),
            pl.BlockSpec((FD, FD), lambda i: (0, 0)),
        ],
        out_specs=[
            pl.BlockSpec((BLK, FD), lambda i: (i, 0)),
            pl.BlockSpec((BLK, FD), lambda i: (i, 0)),
        ],
        out_shape=[
            jax.ShapeDtypeStruct((NPAD, FD), jnp.float32),
            jax.ShapeDtypeStruct((NPAD, FD), jnp.float32),
        ],
    )(deg_p, x_pad, w1)


def _tc2(ap, dinv, b1, w2):
    grid = NPAD // BLK
    return pl.pallas_call(
        _tc2_body,
        grid=(grid,),
        in_specs=[
            pl.BlockSpec((2, BLK, FD), lambda i: (0, i, 0)),
            pl.BlockSpec((BLK, FD), lambda i: (i, 0)),
            pl.BlockSpec((1, FD), lambda i: (0, 0)),
            pl.BlockSpec((FD, FD), lambda i: (0, 0)),
        ],
        out_specs=pl.BlockSpec((BLK, FD), lambda i: (i, 0)),
        out_shape=jax.ShapeDtypeStruct((NPAD, FD), jnp.float32),
    )(ap, dinv, b1, w2)


def _tc3(ap, dinv, b2):
    grid = NPAD // BLK
    return pl.pallas_call(
        _tc3_body,
        grid=(grid,),
        in_specs=[
            pl.BlockSpec((2, BLK, FD), lambda i: (0, i, 0)),
            pl.BlockSpec((BLK, FD), lambda i: (i, 0)),
            pl.BlockSpec((1, FD), lambda i: (0, 0)),
        ],
        out_specs=pl.BlockSpec((BLK, FD), lambda i: (i, 0)),
        out_shape=jax.ShapeDtypeStruct((NPAD, FD), jnp.float32),
    )(ap, dinv, b2)


def kernel(x, edge_index, W1, b1, W2, b2):
    loop = jnp.arange(NN, dtype=jnp.int32)
    fill = jnp.full((EPAD - NE - NN,), NPAD - 1, dtype=jnp.int32)
    src = jnp.concatenate([edge_index[0], loop, fill]).reshape(NW, NCH, CHUNK)
    dst = jnp.concatenate([edge_index[1], loop, fill]).reshape(NW, NCH, CHUNK)

    x_pad = jnp.zeros((NPAD, FD), x.dtype).at[:NN].set(x)
    ones_f = jnp.ones((CHUNK, FD), jnp.float32)
    zero_f = jnp.zeros((RPT, FD), jnp.float32)

    deg_p = _sc_deg(dst, ones_f, zero_f)
    hh1, dinv = _tc1(deg_p, x_pad, W1)
    a1 = _sc_agg(hh1, src, dst, zero_f)
    hh2 = _tc2(a1, dinv, b1.reshape(1, FD), W2)
    a2 = _sc_agg(hh2, src, dst, zero_f)
    out = _tc3(a2, dinv, b2.reshape(1, FD))
    return out[:NN]
